# baseline (speedup 1.0000x reference)
import functools

import jax
import jax.numpy as jnp
from jax import lax
from jax.experimental import pallas as pl
from jax.experimental.pallas import tpu as pltpu
from jax.experimental.pallas import tpu_sc as plsc

N = 2048
D = 1024
FF = 2048
E = 8
CAP = 320
SLOTS = E * CAP
UNC = SLOTS - N
CHUNK = 128
NC = 2
NS = 16
NW = NC * NS
TPW = N // NW
SPW = SLOTS // NW


def _router_body(x_ref, wc_ref, bc_ref,
                 probs_ref, top1_ref, ei_ref, sidx_ref, svals_ref, xs_ref,
                 ndrop_ref, aux_ref):
    x = x_ref[...]
    wc = wc_ref[...]
    logits = lax.dot_general(wc, x, (((1,), (1,)), ((), ())),
                             preferred_element_type=jnp.float32)
    logits = logits + bc_ref[...]
    m = jnp.max(logits, axis=0, keepdims=True)
    ex = jnp.exp(logits - m)
    probs = ex / jnp.sum(ex, axis=0, keepdims=True)
    probs_ref[...] = probs
    top1 = jnp.max(probs, axis=0, keepdims=True)
    top1_ref[...] = top1

    row = lax.broadcasted_iota(jnp.int32, (E, N), 0)
    ai = jnp.min(jnp.where(probs == top1, row, E), axis=0, keepdims=True)
    onehot = (row == ai).astype(jnp.float32)

    ci = lax.broadcasted_iota(jnp.int32, (CHUNK, CHUNK), 0)
    cj = lax.broadcasted_iota(jnp.int32, (CHUNK, CHUNK), 1)
    tri = (ci <= cj).astype(jnp.float32)
    carry = jnp.zeros((E, 1), jnp.float32)
    pris = []
    for i in range(N // CHUNK):
        blk = onehot[:, i * CHUNK:(i + 1) * CHUNK]
        pris.append(carry + lax.dot(blk, tri,
                                    preferred_element_type=jnp.float32))
        carry = carry + jnp.sum(blk, axis=1, keepdims=True)
    pri = jnp.concatenate(pris, axis=1)

    mask = (pri <= float(CAP)).astype(jnp.float32)
    ei = onehot * mask
    ei_ref[...] = ei.astype(jnp.int32)
    kept = jnp.sum(ei, axis=0, keepdims=True) > 0.0
    pr_tok = jnp.sum(onehot * pri, axis=0, keepdims=True)
    slot = ai * CAP + pr_tok.astype(jnp.int32) - 1

    cnt8 = jnp.sum(ei, axis=1, keepdims=True)
    ec8 = float(CAP) - cnt8
    li = lax.broadcasted_iota(jnp.int32, (E, E), 0)
    lj = lax.broadcasted_iota(jnp.int32, (E, E), 1)
    low = (lj < li).astype(jnp.float32)
    offs = lax.dot(low, ec8, preferred_element_type=jnp.float32)

    dropped_f = 1.0 - kept.astype(jnp.float32)
    dcarry = jnp.zeros((1, 1), jnp.float32)
    dr = []
    for i in range(N // CHUNK):
        blk = dropped_f[:, i * CHUNK:(i + 1) * CHUNK]
        dr.append(dcarry + lax.dot(blk, tri,
                                   preferred_element_type=jnp.float32))
        dcarry = dcarry + jnp.sum(blk, axis=1, keepdims=True)
    krank = jnp.concatenate(dr, axis=1) - 1.0

    def empty_slot_for_rank(k):
        m = k.shape[1]
        ge = (offs <= k).astype(jnp.float32)
        e_k = jnp.sum(ge, axis=0, keepdims=True) - 1.0
        oh = (lax.broadcasted_iota(jnp.int32, (E, m), 0).astype(jnp.float32)
              == e_k).astype(jnp.float32)
        cnt_k = jnp.sum(oh * cnt8, axis=0, keepdims=True)
        offs_k = jnp.sum(oh * offs, axis=0, keepdims=True)
        return e_k * float(CAP) + cnt_k + (k - offs_k)

    es_drop = empty_slot_for_rank(krank).astype(jnp.int32)
    fslot = jnp.where(kept, slot, es_drop)
    ku = dcarry + lax.broadcasted_iota(jnp.int32, (1, UNC), 1).astype(jnp.float32)
    es_unc = empty_slot_for_rank(ku).astype(jnp.int32)
    sidx_ref[...] = jnp.concatenate([fslot, es_unc], axis=1)
    tok_iota = lax.broadcasted_iota(jnp.int32, (1, N), 1)
    svals_ref[...] = jnp.concatenate(
        [tok_iota, jnp.full((1, UNC), N, jnp.int32)], axis=1)

    scale = jnp.where(kept, top1, 0.0)
    ident = (ci == cj).astype(jnp.float32)
    cols = []
    for i in range(N // CHUNK):
        blk = scale[:, i * CHUNK:(i + 1) * CHUNK]
        cols.append(lax.dot_general(ident, blk, (((1,), (1,)), ((), ())),
                                    preferred_element_type=jnp.float32))
    scale_col = jnp.concatenate(cols, axis=0)
    xs_ref[...] = x * scale_col

    ndrop_ref[0, 0] = jnp.sum((~kept).astype(jnp.int32))
    fi = jnp.sum(ei, axis=1, keepdims=True) / float(N)
    pi = jnp.sum(probs, axis=1, keepdims=True) / float(N)
    aux_ref[0, 0] = float(E) * jnp.sum(fi * pi)


def _router(x, wc, bc_col):
    return pl.pallas_call(
        _router_body,
        out_shape=(
            jax.ShapeDtypeStruct((E, N), jnp.float32),
            jax.ShapeDtypeStruct((1, N), jnp.float32),
            jax.ShapeDtypeStruct((E, N), jnp.int32),
            jax.ShapeDtypeStruct((1, SLOTS), jnp.int32),
            jax.ShapeDtypeStruct((1, SLOTS), jnp.int32),
            jax.ShapeDtypeStruct((N, D), jnp.float32),
            jax.ShapeDtypeStruct((1, 1), jnp.int32),
            jax.ShapeDtypeStruct((1, 1), jnp.float32),
        ),
        out_specs=(
            pl.BlockSpec(memory_space=pltpu.VMEM),
            pl.BlockSpec(memory_space=pltpu.VMEM),
            pl.BlockSpec(memory_space=pltpu.VMEM),
            pl.BlockSpec(memory_space=pltpu.VMEM),
            pl.BlockSpec(memory_space=pltpu.VMEM),
            pl.BlockSpec(memory_space=pltpu.VMEM),
            pl.BlockSpec(memory_space=pltpu.SMEM),
            pl.BlockSpec(memory_space=pltpu.SMEM),
        ),
    )(x, wc, bc_col)


def _dispatch_body(x_hbm, sidx_hbm, svals_hbm, xbuf_hbm, inv_hbm,
                   idx_v, rows_v, idx2_v, val_v, sem, sem2):
    wid = lax.axis_index("s") * NC + lax.axis_index("c")
    base = wid * TPW
    pltpu.sync_copy(sidx_hbm.at[pl.ds(base, TPW)], idx_v)
    pltpu.sync_copy(x_hbm.at[pl.ds(base, TPW)], rows_v)
    rows_dma = pltpu.async_copy(rows_v, xbuf_hbm.at[idx_v], sem)
    base2 = wid * SPW
    pltpu.sync_copy(sidx_hbm.at[pl.ds(base2, SPW)], idx2_v)
    pltpu.sync_copy(svals_hbm.at[pl.ds(base2, SPW)], val_v)
    pltpu.async_copy(val_v, inv_hbm.at[idx2_v], sem2).wait()
    rows_dma.wait()


def _dispatch(x, sidx, svals):
    mesh = plsc.VectorSubcoreMesh(core_axis_name="c", subcore_axis_name="s")
    return pl.kernel(
        _dispatch_body,
        mesh=mesh,
        out_type=(
            jax.ShapeDtypeStruct((SLOTS, D), jnp.float32),
            jax.ShapeDtypeStruct((SLOTS,), jnp.int32),
        ),
        scratch_types=[
            pltpu.VMEM((TPW,), jnp.int32),
            pltpu.VMEM((TPW, D), jnp.float32),
            pltpu.VMEM((SPW,), jnp.int32),
            pltpu.VMEM((SPW,), jnp.int32),
            pltpu.SemaphoreType.DMA,
            pltpu.SemaphoreType.DMA,
        ],
    )(x, sidx, svals)


def _expert_body(x_ref, wi_ref, wo_ref, inv_ref, o_ref, acc_ref, sem):
    x16 = x_ref[...].astype(jnp.bfloat16)
    h = lax.dot_general(x16, wi_ref[0].astype(jnp.bfloat16),
                        (((1,), (1,)), ((), ())),
                        preferred_element_type=jnp.float32)
    h16 = jnp.maximum(h, 0.0).astype(jnp.bfloat16)
    acc_ref[...] = lax.dot_general(h16, wo_ref[0].astype(jnp.bfloat16),
                                   (((1,), (1,)), ((), ())),
                                   preferred_element_type=jnp.float32)

    def issue(s, carry):
        tok = inv_ref[0, 0, s]

        @pl.when(tok < N)
        def _():
            pltpu.make_async_copy(acc_ref.at[pl.ds(s, 1)],
                                  o_ref.at[pl.ds(tok, 1)], sem).start()
        return carry

    lax.fori_loop(0, CAP, issue, 0, unroll=True)

    def drain(s, carry):
        tok = inv_ref[0, 0, s]

        @pl.when(tok < N)
        def _():
            pltpu.make_async_copy(acc_ref.at[pl.ds(s, 1)],
                                  o_ref.at[pl.ds(tok, 1)], sem).wait()
        return carry

    lax.fori_loop(0, CAP, drain, 0, unroll=True)


def _experts(xbuf, wi, wo, inv3):
    return pl.pallas_call(
        _expert_body,
        grid=(E,),
        in_specs=[
            pl.BlockSpec((CAP, D), lambda e: (e, 0)),
            pl.BlockSpec((1, FF, D), lambda e: (e, 0, 0)),
            pl.BlockSpec((1, D, FF), lambda e: (e, 0, 0)),
            pl.BlockSpec((1, 1, CAP), lambda e: (e, 0, 0),
                         memory_space=pltpu.SMEM),
        ],
        out_specs=pl.BlockSpec(memory_space=pl.ANY),
        out_shape=jax.ShapeDtypeStruct((N, D), jnp.float32),
        scratch_shapes=[
            pltpu.VMEM((CAP, D), jnp.float32),
            pltpu.SemaphoreType.DMA,
        ],
    )(xbuf, wi, wo, inv3)


def kernel(hidden_states, Wc, bc, Wi, Wo):
    b, s, d = hidden_states.shape
    x = hidden_states.reshape(N, D)
    (probs_T, top1_T, ei_T, sidx, svals, xs, ndrop, aux) = _router(
        x, Wc, bc.reshape(E, 1))
    xbuf, inv = _dispatch(xs, sidx.reshape(SLOTS), svals.reshape(SLOTS))
    out = _experts(xbuf, Wi, Wo, inv.reshape(E, 1, CAP))
    return (out.reshape(b, s, d),
            ei_T.T,
            top1_T.reshape(N),
            probs_T.T,
            ndrop.reshape(()),
            aux.reshape(()))


# speedup vs baseline: 1.0034x; 1.0034x over previous
import functools

import jax
import jax.numpy as jnp
from jax import lax
from jax.experimental import pallas as pl
from jax.experimental.pallas import tpu as pltpu
from jax.experimental.pallas import tpu_sc as plsc

N = 2048
D = 1024
FF = 2048
E = 8
CAP = 320
SLOTS = E * CAP
TRASH = SLOTS
XROWS = SLOTS + CAP
CHUNK = 128
NC = 2
NS = 16
NW = NC * NS
TPW = N // NW
FK = 1
FFC = FF // FK


def _router_body(x_ref, wc_ref, bc_ref,
                 probs_ref, top1_ref, ei_ref, fg_ref, fs_ref, scale_ref,
                 rowmask_ref, ndrop_ref, aux_ref):
    x = x_ref[...]
    wc = wc_ref[...]
    logits = lax.dot_general(wc, x, (((1,), (1,)), ((), ())),
                             preferred_element_type=jnp.float32)
    logits = logits + bc_ref[...]
    m = jnp.max(logits, axis=0, keepdims=True)
    ex = jnp.exp(logits - m)
    probs = ex / jnp.sum(ex, axis=0, keepdims=True)
    probs_ref[...] = probs
    top1 = jnp.max(probs, axis=0, keepdims=True)
    top1_ref[...] = top1

    row = lax.broadcasted_iota(jnp.int32, (E, N), 0)
    ai = jnp.min(jnp.where(probs == top1, row, E), axis=0, keepdims=True)
    onehot = (row == ai).astype(jnp.float32)

    ci = lax.broadcasted_iota(jnp.int32, (CHUNK, CHUNK), 0)
    cj = lax.broadcasted_iota(jnp.int32, (CHUNK, CHUNK), 1)
    tri = (ci <= cj).astype(jnp.float32)
    carry = jnp.zeros((E, 1), jnp.float32)
    pris = []
    for i in range(N // CHUNK):
        blk = onehot[:, i * CHUNK:(i + 1) * CHUNK]
        pris.append(carry + lax.dot(blk, tri,
                                    preferred_element_type=jnp.float32))
        carry = carry + jnp.sum(blk, axis=1, keepdims=True)
    pri = jnp.concatenate(pris, axis=1)

    mask = (pri <= float(CAP)).astype(jnp.float32)
    ei = onehot * mask
    ei_ref[...] = ei.astype(jnp.int32)
    kept = jnp.sum(ei, axis=0, keepdims=True) > 0.0
    pr_tok = jnp.sum(onehot * pri, axis=0, keepdims=True)
    slot = ai * CAP + pr_tok.astype(jnp.int32) - 1

    cnt8 = jnp.sum(ei, axis=1, keepdims=True).astype(jnp.int32)
    e_iota = lax.broadcasted_iota(jnp.int32, (E, 1), 0)
    mn = jnp.min(cnt8, axis=0, keepdims=True)
    am = jnp.min(jnp.where(cnt8 == mn, e_iota, E), axis=0, keepdims=True)
    zslot = am * CAP + mn
    fg_ref[...] = jnp.where(kept, slot, zslot)
    fs_ref[...] = jnp.where(kept, slot, TRASH)
    s_iota = lax.broadcasted_iota(jnp.int32, (E, CAP), 1)
    rowmask_ref[...] = (s_iota < cnt8).astype(jnp.float32)

    scale_ref[...] = jnp.where(kept, top1, 0.0)

    ndrop_ref[0, 0] = jnp.sum((~kept).astype(jnp.int32))
    fi = jnp.sum(ei, axis=1, keepdims=True) / float(N)
    pi = jnp.sum(probs, axis=1, keepdims=True) / float(N)
    aux_ref[0, 0] = float(E) * jnp.sum(fi * pi)


def _router(x, wc, bc_col):
    return pl.pallas_call(
        _router_body,
        out_shape=(
            jax.ShapeDtypeStruct((E, N), jnp.float32),
            jax.ShapeDtypeStruct((1, N), jnp.float32),
            jax.ShapeDtypeStruct((E, N), jnp.int32),
            jax.ShapeDtypeStruct((1, N), jnp.int32),
            jax.ShapeDtypeStruct((1, N), jnp.int32),
            jax.ShapeDtypeStruct((1, N), jnp.float32),
            jax.ShapeDtypeStruct((E, CAP), jnp.float32),
            jax.ShapeDtypeStruct((1, 1), jnp.int32),
            jax.ShapeDtypeStruct((1, 1), jnp.float32),
        ),
        out_specs=(
            pl.BlockSpec(memory_space=pltpu.VMEM),
            pl.BlockSpec(memory_space=pltpu.VMEM),
            pl.BlockSpec(memory_space=pltpu.VMEM),
            pl.BlockSpec(memory_space=pltpu.VMEM),
            pl.BlockSpec(memory_space=pltpu.VMEM),
            pl.BlockSpec(memory_space=pltpu.VMEM),
            pl.BlockSpec(memory_space=pltpu.VMEM),
            pl.BlockSpec(memory_space=pltpu.SMEM),
            pl.BlockSpec(memory_space=pltpu.SMEM),
        ),
    )(x, wc, bc_col)


def _dispatch_body(x_hbm, fs_hbm, sc_hbm, xbuf_hbm, ps_hbm,
                   idx_v, rows_v, val_v, sem, sem2):
    wid = lax.axis_index("s") * NC + lax.axis_index("c")
    base = wid * TPW
    pltpu.sync_copy(fs_hbm.at[pl.ds(base, TPW)], idx_v)
    pltpu.sync_copy(x_hbm.at[pl.ds(base, TPW)], rows_v)
    rows_dma = pltpu.async_copy(rows_v, xbuf_hbm.at[idx_v], sem)
    pltpu.sync_copy(sc_hbm.at[pl.ds(base, TPW)], val_v)
    pltpu.async_copy(val_v, ps_hbm.at[idx_v], sem2).wait()
    rows_dma.wait()


def _dispatch(x, fs, scale):
    mesh = plsc.VectorSubcoreMesh(core_axis_name="c", subcore_axis_name="s")
    return pl.kernel(
        _dispatch_body,
        mesh=mesh,
        out_type=(
            jax.ShapeDtypeStruct((XROWS, D), jnp.float32),
            jax.ShapeDtypeStruct((XROWS,), jnp.float32),
        ),
        scratch_types=[
            pltpu.VMEM((TPW,), jnp.int32),
            pltpu.VMEM((TPW, D), jnp.float32),
            pltpu.VMEM((TPW,), jnp.float32),
            pltpu.SemaphoreType.DMA,
            pltpu.SemaphoreType.DMA,
        ],
    )(x, fs, scale)


def _expert_body(x_ref, wi_ref, wo_ref, m_ref, ps_ref, o_ref):
    x16 = x_ref[...].astype(jnp.bfloat16)
    h = lax.dot_general(x16, wi_ref[0].astype(jnp.bfloat16),
                        (((1,), (1,)), ((), ())),
                        preferred_element_type=jnp.float32)
    h16 = jnp.maximum(h, 0.0).astype(jnp.bfloat16)
    acc = lax.dot_general(h16, wo_ref[0].astype(jnp.bfloat16),
                          (((1,), (1,)), ((), ())),
                          preferred_element_type=jnp.float32)
    o_ref[...] = jnp.where(m_ref[...] > 0.0, acc * ps_ref[...], 0.0)


def _experts(xbuf, wi, wo, rowmask_col, ps_col):
    return pl.pallas_call(
        _expert_body,
        grid=(E,),
        in_specs=[
            pl.BlockSpec((CAP, D), lambda e: (e, 0)),
            pl.BlockSpec((1, FF, D), lambda e: (e, 0, 0)),
            pl.BlockSpec((1, D, FF), lambda e: (e, 0, 0)),
            pl.BlockSpec((CAP, 1), lambda e: (e, 0)),
            pl.BlockSpec((CAP, 1), lambda e: (e, 0)),
        ],
        out_specs=pl.BlockSpec((CAP, D), lambda e: (e, 0)),
        out_shape=jax.ShapeDtypeStruct((SLOTS, D), jnp.float32),
    )(xbuf, wi, wo, rowmask_col, ps_col)


def _combine_body(hbuf_hbm, fg_hbm, out_hbm, idx_v, rows_v, sem):
    wid = lax.axis_index("s") * NC + lax.axis_index("c")
    base = wid * TPW
    pltpu.sync_copy(fg_hbm.at[pl.ds(base, TPW)], idx_v)
    pltpu.async_copy(hbuf_hbm.at[idx_v], rows_v, sem).wait()
    pltpu.sync_copy(rows_v, out_hbm.at[pl.ds(base, TPW)])


def _combine(hbuf, fg):
    mesh = plsc.VectorSubcoreMesh(core_axis_name="c", subcore_axis_name="s")
    return pl.kernel(
        _combine_body,
        mesh=mesh,
        out_type=jax.ShapeDtypeStruct((N, D), jnp.float32),
        scratch_types=[
            pltpu.VMEM((TPW,), jnp.int32),
            pltpu.VMEM((TPW, D), jnp.float32),
            pltpu.SemaphoreType.DMA,
        ],
    )(hbuf, fg)


def kernel(hidden_states, Wc, bc, Wi, Wo):
    b, s, d = hidden_states.shape
    x = hidden_states.reshape(N, D)
    (probs_T, top1_T, ei_T, fg, fs, scale_T, rowmask, ndrop, aux) = _router(
        x, Wc, bc.reshape(E, 1))
    xbuf, pscale = _dispatch(x, fs.reshape(N), scale_T.reshape(N))
    hbuf = _experts(xbuf, Wi, Wo, rowmask.reshape(SLOTS, 1),
                    pscale[:SLOTS].reshape(SLOTS, 1))
    out = _combine(hbuf, fg.reshape(N))
    return (out.reshape(b, s, d),
            ei_T.T,
            top1_T.reshape(N),
            probs_T.T,
            ndrop.reshape(()),
            aux.reshape(()))


# speedup vs baseline: 1.2064x; 1.2023x over previous
import functools

import jax
import jax.numpy as jnp
from jax import lax
from jax.experimental import pallas as pl
from jax.experimental.pallas import tpu as pltpu
from jax.experimental.pallas import tpu_sc as plsc

N = 2048
D = 1024
FF = 2048
E = 8
CAP = 320
SLOTS = E * CAP
TRASH = SLOTS
XROWS = SLOTS + CAP
CHUNK = 128
NC = 2
NS = 16
NW = NC * NS
TPW = N // NW
FK = 1
FFC = FF // FK


def _router_body(x_ref, wc_ref, bc_ref,
                 probs_ref, top1_ref, ei_ref, fg_ref, fs_ref, xs_ref,
                 rowmask_ref, ndrop_ref, aux_ref):
    x = x_ref[...]
    wc = wc_ref[...]
    logits = lax.dot_general(wc, x, (((1,), (1,)), ((), ())),
                             preferred_element_type=jnp.float32)
    logits = logits + bc_ref[...]
    m = jnp.max(logits, axis=0, keepdims=True)
    ex = jnp.exp(logits - m)
    probs = ex / jnp.sum(ex, axis=0, keepdims=True)
    probs_ref[...] = probs
    top1 = jnp.max(probs, axis=0, keepdims=True)
    top1_ref[...] = top1

    row = lax.broadcasted_iota(jnp.int32, (E, N), 0)
    ai = jnp.min(jnp.where(probs == top1, row, E), axis=0, keepdims=True)
    onehot = (row == ai).astype(jnp.float32)

    ci = lax.broadcasted_iota(jnp.int32, (CHUNK, CHUNK), 0)
    cj = lax.broadcasted_iota(jnp.int32, (CHUNK, CHUNK), 1)
    tri = (ci <= cj).astype(jnp.float32)
    carry = jnp.zeros((E, 1), jnp.float32)
    pris = []
    for i in range(N // CHUNK):
        blk = onehot[:, i * CHUNK:(i + 1) * CHUNK]
        pris.append(carry + lax.dot(blk, tri,
                                    preferred_element_type=jnp.float32))
        carry = carry + jnp.sum(blk, axis=1, keepdims=True)
    pri = jnp.concatenate(pris, axis=1)

    mask = (pri <= float(CAP)).astype(jnp.float32)
    ei = onehot * mask
    ei_ref[...] = ei.astype(jnp.int32)
    kept = jnp.sum(ei, axis=0, keepdims=True) > 0.0
    pr_tok = jnp.sum(onehot * pri, axis=0, keepdims=True)
    slot = ai * CAP + pr_tok.astype(jnp.int32) - 1

    cnt8 = jnp.sum(ei, axis=1, keepdims=True).astype(jnp.int32)
    e_iota = lax.broadcasted_iota(jnp.int32, (E, 1), 0)
    mn = jnp.min(cnt8, axis=0, keepdims=True)
    am = jnp.min(jnp.where(cnt8 == mn, e_iota, E), axis=0, keepdims=True)
    zslot = am * CAP + mn
    fg_ref[...] = jnp.where(kept, slot, zslot)
    fs_ref[...] = jnp.where(kept, slot, TRASH)
    s_iota = lax.broadcasted_iota(jnp.int32, (E, CAP), 1)
    rowmask_ref[...] = (s_iota < cnt8).astype(jnp.float32)

    scale = jnp.where(kept, top1, 0.0)
    ident = (ci == cj).astype(jnp.float32)
    cols = []
    for i in range(N // CHUNK):
        blk = scale[:, i * CHUNK:(i + 1) * CHUNK]
        cols.append(lax.dot_general(ident, blk, (((1,), (1,)), ((), ())),
                                    preferred_element_type=jnp.float32))
    scale_col = jnp.concatenate(cols, axis=0)
    xs_ref[...] = x * scale_col

    ndrop_ref[0, 0] = jnp.sum((~kept).astype(jnp.int32))
    fi = jnp.sum(ei, axis=1, keepdims=True) / float(N)
    pi = jnp.sum(probs, axis=1, keepdims=True) / float(N)
    aux_ref[0, 0] = float(E) * jnp.sum(fi * pi)


def _router(x, wc, bc_col):
    return pl.pallas_call(
        _router_body,
        out_shape=(
            jax.ShapeDtypeStruct((E, N), jnp.float32),
            jax.ShapeDtypeStruct((1, N), jnp.float32),
            jax.ShapeDtypeStruct((E, N), jnp.int32),
            jax.ShapeDtypeStruct((1, N), jnp.int32),
            jax.ShapeDtypeStruct((1, N), jnp.int32),
            jax.ShapeDtypeStruct((N, D), jnp.float32),
            jax.ShapeDtypeStruct((E, CAP), jnp.float32),
            jax.ShapeDtypeStruct((1, 1), jnp.int32),
            jax.ShapeDtypeStruct((1, 1), jnp.float32),
        ),
        out_specs=(
            pl.BlockSpec(memory_space=pltpu.VMEM),
            pl.BlockSpec(memory_space=pltpu.VMEM),
            pl.BlockSpec(memory_space=pltpu.VMEM),
            pl.BlockSpec(memory_space=pltpu.VMEM),
            pl.BlockSpec(memory_space=pltpu.VMEM),
            pl.BlockSpec(memory_space=pltpu.VMEM),
            pl.BlockSpec(memory_space=pltpu.VMEM),
            pl.BlockSpec(memory_space=pltpu.SMEM),
            pl.BlockSpec(memory_space=pltpu.SMEM),
        ),
    )(x, wc, bc_col)


def _dispatch_body(x_hbm, fs_hbm, xbuf_hbm, idx_v, rows_v, sem):
    wid = lax.axis_index("s") * NC + lax.axis_index("c")
    base = wid * TPW
    pltpu.sync_copy(fs_hbm.at[pl.ds(base, TPW)], idx_v)
    pltpu.sync_copy(x_hbm.at[pl.ds(base, TPW)], rows_v)
    pltpu.async_copy(rows_v, xbuf_hbm.at[idx_v], sem).wait()


def _dispatch(x, fs):
    mesh = plsc.VectorSubcoreMesh(core_axis_name="c", subcore_axis_name="s")
    return pl.kernel(
        _dispatch_body,
        mesh=mesh,
        out_type=jax.ShapeDtypeStruct((XROWS, D), jnp.float32),
        scratch_types=[
            pltpu.VMEM((TPW,), jnp.int32),
            pltpu.VMEM((TPW, D), jnp.float32),
            pltpu.SemaphoreType.DMA,
        ],
    )(x, fs)


def _expert_body(x_ref, wi_ref, wo_ref, m_ref, o_ref):
    k = pl.program_id(1)
    x16 = x_ref[...].astype(jnp.bfloat16)
    h = lax.dot_general(x16, wi_ref[0].astype(jnp.bfloat16),
                        (((1,), (1,)), ((), ())),
                        preferred_element_type=jnp.float32)
    h16 = jnp.maximum(h, 0.0).astype(jnp.bfloat16)
    acc = lax.dot_general(h16, wo_ref[0].astype(jnp.bfloat16),
                          (((1,), (1,)), ((), ())),
                          preferred_element_type=jnp.float32)
    acc = jnp.where(m_ref[...] > 0.0, acc, 0.0)

    @pl.when(k == 0)
    def _():
        o_ref[...] = acc

    @pl.when(k != 0)
    def _():
        o_ref[...] = o_ref[...] + acc


def _experts(xbuf, wi, wo, rowmask_col):
    return pl.pallas_call(
        _expert_body,
        grid=(E, FK),
        in_specs=[
            pl.BlockSpec((CAP, D), lambda e, k: (e, 0)),
            pl.BlockSpec((1, FFC, D), lambda e, k: (e, k, 0)),
            pl.BlockSpec((1, D, FFC), lambda e, k: (e, 0, k)),
            pl.BlockSpec((CAP, 1), lambda e, k: (e, 0)),
        ],
        out_specs=pl.BlockSpec((CAP, D), lambda e, k: (e, 0)),
        out_shape=jax.ShapeDtypeStruct((SLOTS, D), jnp.float32),
    )(xbuf, wi, wo, rowmask_col)


def _combine_body(hbuf_hbm, fg_hbm, out_hbm, idx_v, rows_v, sem):
    wid = lax.axis_index("s") * NC + lax.axis_index("c")
    base = wid * TPW
    pltpu.sync_copy(fg_hbm.at[pl.ds(base, TPW)], idx_v)
    pltpu.async_copy(hbuf_hbm.at[idx_v], rows_v, sem).wait()
    pltpu.sync_copy(rows_v, out_hbm.at[pl.ds(base, TPW)])


def _combine(hbuf, fg):
    mesh = plsc.VectorSubcoreMesh(core_axis_name="c", subcore_axis_name="s")
    return pl.kernel(
        _combine_body,
        mesh=mesh,
        out_type=jax.ShapeDtypeStruct((N, D), jnp.float32),
        scratch_types=[
            pltpu.VMEM((TPW,), jnp.int32),
            pltpu.VMEM((TPW, D), jnp.float32),
            pltpu.SemaphoreType.DMA,
        ],
    )(hbuf, fg)


def kernel(hidden_states, Wc, bc, Wi, Wo):
    b, s, d = hidden_states.shape
    x = hidden_states.reshape(N, D)
    (probs_T, top1_T, ei_T, fg, fs, xs, rowmask, ndrop, aux) = _router(
        x, Wc, bc.reshape(E, 1))
    xbuf = _dispatch(xs, fs.reshape(N))
    hbuf = _experts(xbuf, Wi, Wo, rowmask.reshape(SLOTS, 1))
    out = _combine(hbuf, fg.reshape(N))
    return (out.reshape(b, s, d),
            ei_T.T,
            top1_T.reshape(N),
            probs_T.T,
            ndrop.reshape(()),
            aux.reshape(()))
